# baseline (speedup 1.0000x reference)
import functools

import jax
import jax.numpy as jnp
from jax import lax
from jax.experimental import pallas as pl
from jax.experimental.pallas import tpu as pltpu
from jax.experimental.pallas import tpu_sc as plsc

EMBED = 32
D_OUT = 70
B, L = 16384, 200
N = B * L
NC, NS = 2, 16
NW = NC * NS
N_W = N // NW
NB = 2
CH = NB * L
NCHUNK = N_W // CH
GRP = CH // 16

SIN_C = (6.283183465409584, -41.34148025958733, 81.59765524711814,
         -76.59489967393353, 41.26979637356445, -12.372272029175647)
COS_C = (0.9999999922855516, -19.739205552336067, 64.939172135788,
         -85.45116383102753, 60.176212682457354, -26.000455681228082,
         6.575502264032736)


def _horner(coeffs, z):
    r = jnp.float32(coeffs[-1])
    for c in coeffs[-2::-1]:
        r = r * z + jnp.float32(c)
    return r


def _sc_body(hour_hbm, wday_hbm, nt_hbm, comb_hbm, out_hbm,
             comb_v, h_v, w_v, t_v, out0_v, out1_v, sem0, sem1):
    wid = lax.axis_index("s") * NC + lax.axis_index("c")
    pltpu.sync_copy(comb_hbm, comb_v)
    iota16 = lax.iota(jnp.int32, 16)

    def do_chunk(g, gg, out_v, sem_out):
        base = wid * N_W + g * CH
        brow = (wid * N_W + g * CH) // L
        pltpu.sync_copy(hour_hbm.at[pl.ds(base, CH)], h_v)
        pltpu.sync_copy(wday_hbm.at[pl.ds(base, CH)], w_v)
        pltpu.sync_copy(nt_hbm.at[pl.ds(base, CH)], t_v)

        def grp_body(j, _):
            h = h_v[pl.ds(j * 16, 16)]
            w = w_v[pl.ds(j * 16, 16)]
            t = t_v[pl.ds(j * 16, 16)]
            cidx = (h * 7 + w) * (2 * EMBED)
            elem = j * 16 + iota16
            ob0 = elem // L
            ob1 = elem - ob0 * L

            @plsc.parallel_loop(0, 2 * EMBED, unroll=64)
            def _(d):
                val = plsc.load_gather(comb_v, [cidx + d])
                dcol = jnp.full((16,), d, jnp.int32)
                plsc.store_scatter(out_v, [ob0, ob1, dcol], val)
            x = t - lax.convert_element_type(
                lax.convert_element_type(t + 0.5, jnp.int32), jnp.float32)
            z = x * x
            s1 = x * _horner(SIN_C, z)
            c1 = _horner(COS_C, z)
            s2 = 2.0 * s1 * c1
            c2 = 1.0 - 2.0 * s1 * s1
            s4 = 2.0 * s2 * c2
            c4 = 1.0 - 2.0 * s2 * s2
            for k, val in enumerate((s1, c1, s2, c2, s4, c4)):
                kcol = jnp.full((16,), 2 * EMBED + k, jnp.int32)
                plsc.store_scatter(out_v, [ob0, ob1, kcol], val)
            return 0

        @pl.when(gg >= 1)
        def _():
            pltpu.make_async_copy(
                out_v,
                out_hbm.at[pl.ds(brow - 2 * NB, NB), :, :],
                sem_out).wait()

        @plsc.parallel_loop(0, GRP)
        def _(j):
            grp_body(j, 0)
        pltpu.make_async_copy(
            out_v,
            out_hbm.at[pl.ds(brow, NB), :, :],
            sem_out).start()

    def chunk_pair(gg, _):
        do_chunk(gg * 2, gg, out0_v, sem0)
        do_chunk(gg * 2 + 1, gg, out1_v, sem1)
        return 0

    lax.fori_loop(0, NCHUNK // 2, chunk_pair, 0)
    for buf, (out_v, sem_out) in enumerate(((out0_v, sem0), (out1_v, sem1))):
        g = NCHUNK - 2 + buf
        brow = (wid * N_W + g * CH) // L
        pltpu.make_async_copy(
            out_v,
            out_hbm.at[pl.ds(brow, NB), :, :],
            sem_out).wait()


@functools.partial(jax.jit, static_argnums=())
def _encode(hour_f, wday_f, nt_f, comb_flat):
    mesh = plsc.VectorSubcoreMesh(core_axis_name="c", subcore_axis_name="s")
    fn = pl.kernel(
        _sc_body,
        mesh=mesh,
        compiler_params=pltpu.CompilerParams(
            needs_layout_passes=False, use_tc_tiling_on_sc=True),
        out_type=jax.ShapeDtypeStruct((B, L, D_OUT), jnp.float32),
        scratch_types=[
            pltpu.VMEM((24 * 7 * 2 * EMBED,), jnp.float32),
            pltpu.VMEM((CH,), jnp.int32),
            pltpu.VMEM((CH,), jnp.int32),
            pltpu.VMEM((CH,), jnp.float32),
            pltpu.VMEM((NB, L, D_OUT), jnp.float32),
            pltpu.VMEM((NB, L, D_OUT), jnp.float32),
            pltpu.SemaphoreType.DMA,
            pltpu.SemaphoreType.DMA,
        ],
    )
    return fn(hour_f, wday_f, nt_f, comb_flat)


def kernel(hour, weekday, norm_time, hour_table, weekday_table):
    hour_f = hour.reshape(N).astype(jnp.int32)
    wday_f = weekday.reshape(N).astype(jnp.int32)
    nt_f = norm_time.reshape(N)
    comb_flat = jnp.concatenate([
        jnp.broadcast_to(hour_table[:, None, :], (24, 7, EMBED)),
        jnp.broadcast_to(weekday_table[None, :, :], (24, 7, EMBED)),
    ], axis=-1).reshape(24 * 7 * 2 * EMBED)
    return _encode(hour_f, wday_f, nt_f, comb_flat)


# speedup vs baseline: 1.9518x; 1.9518x over previous
import functools

import jax
import jax.numpy as jnp
from jax import lax
from jax.experimental import pallas as pl
from jax.experimental.pallas import tpu as pltpu
from jax.experimental.pallas import tpu_sc as plsc

EMBED = 32
TW = 2 * EMBED
D_OUT = 70
B, L = 16384, 200
N = B * L
NC, NS = 2, 16
NW = NC * NS
WB = B // NW
NBT = WB // 128
NBLK = L // 8
UPB = 8 * NBT

SIN_C = (6.283183465409584, -41.34148025958733, 81.59765524711814,
         -76.59489967393353, 41.26979637356445, -12.372272029175647)
COS_C = (0.9999999922855516, -19.739205552336067, 64.939172135788,
         -85.45116383102753, 60.176212682457354, -26.000455681228082,
         6.575502264032736)


def _horner(coeffs, z):
    r = jnp.float32(coeffs[-1])
    for c in coeffs[-2::-1]:
        r = r * z + jnp.float32(c)
    return r


def _sc_body(hour_hbm, wday_hbm, nt_hbm, comb_hbm, out_hbm,
             comb_v, h0_v, w0_v, t0_v, h1_v, w1_v, t1_v,
             out0_v, out1_v, semi0, semi1, sem0, sem1):
    wid = lax.axis_index("s") * NC + lax.axis_index("c")
    wb0 = wid * WB
    pltpu.sync_copy(comb_hbm, comb_v)
    inbufs = ((h0_v, w0_v, t0_v, semi0), (h1_v, w1_v, t1_v, semi1))
    outbufs = ((out0_v, sem0), (out1_v, sem1))

    def in_copies(blk, pb):
        h_v, w_v, t_v, semi = inbufs[pb]
        rows = pl.ds(blk * 8, 8)
        cols = pl.ds(wb0, WB)
        return (
            pltpu.make_async_copy(hour_hbm.at[rows, cols], h_v, semi),
            pltpu.make_async_copy(wday_hbm.at[rows, cols], w_v, semi),
            pltpu.make_async_copy(nt_hbm.at[rows, cols], t_v, semi),
        )

    def do_unit(u, blk, i, pb, par):
        h_v, w_v, t_v, _ = inbufs[pb]
        out_v, sem_out = outbufs[par]
        l_off = i // NBT
        bt = i - l_off * NBT
        l = blk * 8 + l_off

        @pl.when(u >= 2)
        def _():
            pltpu.make_async_copy(
                out_v, out_hbm.at[:, 0, pl.ds(0, 128)], sem_out).wait()

        def grp_body(j, _):
            col = pl.ds(bt * 128 + j * 16, 16)
            h = h_v[l_off, col]
            w = w_v[l_off, col]
            t = t_v[l_off, col]
            cidx = (h * 7 + w) * TW

            @plsc.parallel_loop(0, TW, unroll=64)
            def _(d):
                val = plsc.load_gather(comb_v, [cidx + d])
                out_v[d, pl.ds(j * 16, 16)] = val

            x = t - lax.convert_element_type(
                lax.convert_element_type(t + 0.5, jnp.int32), jnp.float32)
            z = x * x
            s1 = x * _horner(SIN_C, z)
            c1 = _horner(COS_C, z)
            s2 = 2.0 * s1 * c1
            c2 = 1.0 - 2.0 * s1 * s1
            s4 = 2.0 * s2 * c2
            c4 = 1.0 - 2.0 * s2 * s2
            for k, val in enumerate((s1, c1, s2, c2, s4, c4)):
                out_v[TW + k, pl.ds(j * 16, 16)] = val
            return 0

        @plsc.parallel_loop(0, 8)
        def _(j):
            grp_body(j, 0)

        pltpu.make_async_copy(
            out_v,
            out_hbm.at[:, l, pl.ds(wb0 + bt * 128, 128)],
            sem_out).start()

    def run_block(blk, pb):
        for c in in_copies(blk, pb):
            c.wait()

        def unit_pair(ip, _):
            i0 = ip * 2
            do_unit(blk * UPB + i0, blk, i0, pb, 0)
            do_unit(blk * UPB + i0 + 1, blk, i0 + 1, pb, 1)
            return 0

        lax.fori_loop(0, UPB // 2, unit_pair, 0)

    for c in in_copies(0, 0):
        c.start()
    for c in in_copies(1, 1):
        c.start()

    def blk_pair(bp, _):
        for pb in range(2):
            blk = bp * 2 + pb
            run_block(blk, pb)

            @pl.when(blk + 2 < NBLK)
            def _():
                for c in in_copies(blk + 2, pb):
                    c.start()
        return 0

    lax.fori_loop(0, (NBLK - 1) // 2, blk_pair, 0)
    run_block(NBLK - 1, 0)

    for par in range(2):
        out_v, sem_out = outbufs[par]
        pltpu.make_async_copy(
            out_v, out_hbm.at[:, 0, pl.ds(0, 128)], sem_out).wait()


@functools.partial(jax.jit, static_argnums=())
def _encode(hour_t, wday_t, nt_t, comb_flat):
    mesh = plsc.VectorSubcoreMesh(core_axis_name="c", subcore_axis_name="s")
    fn = pl.kernel(
        _sc_body,
        mesh=mesh,
        compiler_params=pltpu.CompilerParams(
            needs_layout_passes=False, use_tc_tiling_on_sc=True),
        out_type=jax.ShapeDtypeStruct((D_OUT, L, B), jnp.float32),
        scratch_types=[
            pltpu.VMEM((24 * 7 * TW,), jnp.float32),
            pltpu.VMEM((8, WB), jnp.int32),
            pltpu.VMEM((8, WB), jnp.int32),
            pltpu.VMEM((8, WB), jnp.float32),
            pltpu.VMEM((8, WB), jnp.int32),
            pltpu.VMEM((8, WB), jnp.int32),
            pltpu.VMEM((8, WB), jnp.float32),
            pltpu.VMEM((D_OUT, 128), jnp.float32),
            pltpu.VMEM((D_OUT, 128), jnp.float32),
            pltpu.SemaphoreType.DMA,
            pltpu.SemaphoreType.DMA,
            pltpu.SemaphoreType.DMA,
            pltpu.SemaphoreType.DMA,
        ],
    )
    return fn(hour_t, wday_t, nt_t, comb_flat)


def kernel(hour, weekday, norm_time, hour_table, weekday_table):
    hour_t = hour.T.astype(jnp.int32)
    wday_t = weekday.T.astype(jnp.int32)
    nt_t = norm_time.T
    comb_flat = jnp.concatenate([
        jnp.broadcast_to(hour_table[:, None, :], (24, 7, EMBED)),
        jnp.broadcast_to(weekday_table[None, :, :], (24, 7, EMBED)),
    ], axis=-1).reshape(24 * 7 * TW)
    out_dlb = _encode(hour_t, wday_t, nt_t, comb_flat)
    return jnp.transpose(out_dlb, (2, 1, 0))
